# Initial kernel scaffold; baseline (speedup 1.0000x reference)
import functools

import jax
import jax.numpy as jnp
from jax import lax
from jax.experimental import pallas as pl
from jax.experimental.pallas import tpu as pltpu
from jax.experimental.pallas import tpu_sc as plsc

_N = 10000
_E = 320000
_G = 64
_D = 128
_NC = 2
_NS = 16
_NW = _NC * _NS
_EPW = _E // _NW
_ECH = 125
_ENC = _EPW // _ECH
_NPAD = 10240
_NPW = _NPAD // _NW
_NCH = 80
_NNC = _NPW // _NCH
_RPS = _N // _NS
_DW = 16

_sc_mesh = plsc.VectorSubcoreMesh(core_axis_name="c", subcore_axis_name="s")


@functools.partial(
    pl.kernel,
    out_type=(
        jax.ShapeDtypeStruct((_NC, 2, _N, _DW), jnp.float32),
        jax.ShapeDtypeStruct((_NPAD, 64), jnp.float32),
        jax.ShapeDtypeStruct((_NPAD, 64), jnp.float32),
    ),
    mesh=_sc_mesh,
    scratch_types=[
        pltpu.VMEM((_ENC, _ECH), jnp.int32),
        pltpu.VMEM((_ENC, _ECH), jnp.int32),
        pltpu.VMEM((_NNC, _NCH), jnp.int32),
        pltpu.VMEM((_NCH, 64), jnp.float32),
        pltpu.VMEM((_ECH, _DW), jnp.float32),
        pltpu.VMEM_SHARED((_N, _DW), jnp.float32),
        pltpu.VMEM_SHARED((_N, _DW), jnp.float32),
        pltpu.SemaphoreType.DMA,
    ],
)
def _sc_prologue(ei, nidp, zp, emb, ztab, ones_h, zer16, degs, xa, xb,
                 sidx, didx, gidx, grows, ones_v, ds_sh, dd_sh, sem):
    c = lax.axis_index("c")
    s = lax.axis_index("s")
    w = c * _NS + s
    pltpu.sync_copy(zer16, ds_sh.at[pl.ds(s * _RPS, _RPS)])
    pltpu.sync_copy(zer16, dd_sh.at[pl.ds(s * _RPS, _RPS)])
    pltpu.sync_copy(ones_h, ones_v)
    pltpu.sync_copy(ei.at[0, w], sidx)
    pltpu.sync_copy(ei.at[1, w], didx)
    plsc.subcore_barrier()

    @pl.loop(0, _ENC)
    def _deg(j):
        pltpu.sync_copy(ones_v, ds_sh.at[sidx.at[j]], add=True)
        pltpu.sync_copy(ones_v, dd_sh.at[didx.at[j]], add=True)

    pltpu.sync_copy(nidp.at[w], gidx)

    @pl.loop(0, _NNC)
    def _ga(j):
        pltpu.async_copy(emb.at[gidx.at[j]], grows, sem).wait()
        pltpu.sync_copy(grows, xa.at[pl.ds(w * _NPW + j * _NCH, _NCH)])

    pltpu.sync_copy(zp.at[w], gidx)

    @pl.loop(0, _NNC)
    def _gb(j):
        pltpu.async_copy(ztab.at[gidx.at[j]], grows, sem).wait()
        pltpu.sync_copy(grows, xb.at[pl.ds(w * _NPW + j * _NCH, _NCH)])

    plsc.subcore_barrier()
    pltpu.sync_copy(ds_sh.at[pl.ds(s * _RPS, _RPS)],
                    degs.at[c, 0, pl.ds(s * _RPS, _RPS)])
    pltpu.sync_copy(dd_sh.at[pl.ds(s * _RPS, _RPS)],
                    degs.at[c, 1, pl.ds(s * _RPS, _RPS)])


@functools.partial(
    pl.kernel,
    out_type=jax.ShapeDtypeStruct((_NC, _N, _D), jnp.float32),
    mesh=_sc_mesh,
    scratch_types=[
        pltpu.VMEM((_ENC, _ECH), jnp.int32),
        pltpu.VMEM((_ENC, _ECH), jnp.int32),
        pltpu.VMEM((_ECH, _D), jnp.float32),
        pltpu.VMEM((_ECH, _D), jnp.float32),
        pltpu.VMEM_SHARED((_N, _D), jnp.float32),
        pltpu.SemaphoreType.DMA,
        pltpu.SemaphoreType.DMA,
    ],
)
def _sc_prop(h, ei, zer128, accp, sidx, didx, rows0, rows1, acc_sh, sem0, sem1):
    c = lax.axis_index("c")
    s = lax.axis_index("s")
    w = c * _NS + s
    pltpu.sync_copy(zer128, acc_sh.at[pl.ds(s * _RPS, _RPS)])
    pltpu.sync_copy(ei.at[0, w], sidx)
    pltpu.sync_copy(ei.at[1, w], didx)
    plsc.subcore_barrier()

    pltpu.async_copy(h.at[sidx.at[0]], rows0, sem0).wait()

    @pl.loop(0, _ENC - 1)
    def _edges(j):
        even = j % 2 == 0

        @pl.when(even)
        def _():
            pltpu.async_copy(h.at[sidx.at[j + 1]], rows1, sem1).wait()
            pltpu.sync_copy(rows0, acc_sh.at[didx.at[j]], add=True)

        @pl.when(jnp.logical_not(even))
        def _():
            pltpu.async_copy(h.at[sidx.at[j + 1]], rows0, sem0).wait()
            pltpu.sync_copy(rows1, acc_sh.at[didx.at[j]], add=True)

    last_even = (_ENC - 1) % 2 == 0

    @pl.when(last_even)
    def _le():
        pltpu.sync_copy(rows0, acc_sh.at[didx.at[_ENC - 1]], add=True)

    @pl.when(not last_even)
    def _lo():
        pltpu.sync_copy(rows1, acc_sh.at[didx.at[_ENC - 1]], add=True)

    plsc.subcore_barrier()
    pltpu.sync_copy(acc_sh.at[pl.ds(s * _RPS, _RPS)],
                    accp.at[c, pl.ds(s * _RPS, _RPS)])


def _norms(degs):
    ns = lax.rsqrt(jnp.maximum((degs[0, 0] + degs[1, 0])[:, 0:1], 1.0))
    nd = lax.rsqrt(jnp.maximum((degs[0, 1] + degs[1, 1])[:, 0:1], 1.0))
    return ns, nd


def _tc_h0_body(xa_ref, xb_ref, degs_ref, h0_ref):
    ns, _ = _norms(degs_ref[...])
    x = jnp.concatenate([xa_ref[...], xb_ref[...]], axis=1)
    h0_ref[...] = x * ns


def _tc_layer_body(accp_ref, degs_ref, w_ref, b_ref, h_ref):
    ns, nd = _norms(degs_ref[...])
    agg = (accp_ref[0] + accp_ref[1]) * nd
    o = jnp.dot(agg, w_ref[...], preferred_element_type=jnp.float32) + b_ref[...]
    h_ref[...] = jnp.maximum(o, 0.0) * ns


def _tc_final_body(accp_ref, degs_ref, gid_ref, w3_ref, b3_ref,
                   l1w_ref, l1b_ref, l2w_ref, l2b_ref, out_ref):
    _, nd = _norms(degs_ref[...])
    agg = (accp_ref[0] + accp_ref[1]) * nd
    gio = lax.broadcasted_iota(jnp.int32, (_G, _N), 0)
    onehot = (gid_ref[...][None, :] == gio).astype(jnp.float32)
    pooled_agg = jnp.dot(onehot, agg, preferred_element_type=jnp.float32)
    counts = jnp.sum(onehot, axis=1, keepdims=True)
    pooled = (jnp.dot(pooled_agg, w3_ref[...], preferred_element_type=jnp.float32)
              + counts * b3_ref[...][None, :])
    hh = jnp.maximum(
        jnp.dot(pooled, l1w_ref[...], preferred_element_type=jnp.float32)
        + l1b_ref[...][None, :], 0.0)
    out_ref[...] = (jnp.dot(hh, l2w_ref[...], preferred_element_type=jnp.float32)
                    + l2b_ref[...][None, :])


def _tc_h0(xa, xb, degs):
    return pl.pallas_call(
        _tc_h0_body,
        out_shape=jax.ShapeDtypeStruct((_N, _D), jnp.float32),
    )(xa, xb, degs)


def _tc_layer(accp, degs, w, b):
    return pl.pallas_call(
        _tc_layer_body,
        out_shape=jax.ShapeDtypeStruct((_N, _D), jnp.float32),
    )(accp, degs, w, b)


def _tc_final(accp, degs, gid, w3, b3, l1w, l1b, l2w, l2b):
    return pl.pallas_call(
        _tc_final_body,
        out_shape=jax.ShapeDtypeStruct((_G, 200), jnp.float32),
    )(accp, degs, gid, w3, b3, l1w, l1b, l2w, l2b)


def kernel(edge_index, nid, z, graph_ids, init_embed, z_table,
           W1, b1, W2, b2, W3, b3, l1W, l1b, l2W, l2b):
    ei = edge_index.astype(jnp.int32).reshape(2, _NW, _ENC, _ECH)
    nidp = jnp.pad(nid.astype(jnp.int32), (0, _NPAD - _N)).reshape(_NW, _NNC, _NCH)
    zp = jnp.pad(z.astype(jnp.int32), (0, _NPAD - _N)).reshape(_NW, _NNC, _NCH)
    ones_h = jnp.ones((_ECH, _DW), jnp.float32)
    zer16 = jnp.zeros((_RPS, _DW), jnp.float32)
    zer128 = jnp.zeros((_RPS, _D), jnp.float32)

    degs, xa, xb = _sc_prologue(ei, nidp, zp, init_embed, z_table, ones_h, zer16)
    h0 = _tc_h0(xa[:_N], xb[:_N], degs)
    accp1 = _sc_prop(h0, ei, zer128)
    h1 = _tc_layer(accp1, degs, W1, b1)
    accp2 = _sc_prop(h1, ei, zer128)
    h2 = _tc_layer(accp2, degs, W2, b2)
    accp3 = _sc_prop(h2, ei, zer128)
    return _tc_final(accp3, degs, graph_ids.astype(jnp.int32),
                     W3, b3, l1W, l1b, l2W, l2b)


# speedup vs baseline: 2.0591x; 2.0591x over previous
import functools

import jax
import jax.numpy as jnp
from jax import lax
from jax.experimental import pallas as pl
from jax.experimental.pallas import tpu as pltpu
from jax.experimental.pallas import tpu_sc as plsc

_N = 10000
_E = 320000
_G = 64
_D = 128
_NC = 2
_NS = 16
_NW = _NC * _NS
_EPW = _E // _NW
_ECH = 125
_ENC = _EPW // _ECH
_NP = 10240
_NPW = _NP // _NW
_NCH = 80
_NNC = _NPW // _NCH
_RPS = _NP // _NS
_HN = _NP // _NC
_EP = 327680
_ECH2 = 128
_ENC2 = _EP // _NS // _ECH2
_ACC = _HN + 1024
_AZS = _ACC // _NS
_XPS = _HN // _NS

_sc_mesh = plsc.VectorSubcoreMesh(core_axis_name="c", subcore_axis_name="s")


@functools.partial(
    pl.kernel,
    out_type=(
        jax.ShapeDtypeStruct((_NC, 2, _NP), jnp.float32),
        jax.ShapeDtypeStruct((_NP, _D), jnp.float32),
        jax.ShapeDtypeStruct((_NP, _D), jnp.float32),
        jax.ShapeDtypeStruct((_NC, _NS, _ENC2, _ECH2), jnp.int32),
    ),
    mesh=_sc_mesh,
    scratch_types=[
        pltpu.VMEM((_ENC, _ECH), jnp.int32),
        pltpu.VMEM((_ENC, _ECH), jnp.int32),
        pltpu.VMEM((_NNC, _NCH), jnp.int32),
        pltpu.VMEM((_NCH, _D), jnp.float32),
        pltpu.VMEM((_ECH,), jnp.float32),
        pltpu.VMEM((_ENC2, _ECH2), jnp.int32),
        pltpu.VMEM_SHARED((_NP,), jnp.float32),
        pltpu.VMEM_SHARED((_NP,), jnp.float32),
        pltpu.SemaphoreType.DMA,
    ],
)
def _sc_prologue(ei, nidp, zp, dpad, emb, ztab, ones_h, zer1,
                 degs, xa, xb, didx_t,
                 sidx, didx, gidx, grows, ones_v, tidx, ds_sh, dd_sh, sem):
    c = lax.axis_index("c")
    s = lax.axis_index("s")
    w = c * _NS + s
    pltpu.sync_copy(zer1, ds_sh.at[pl.ds(s * _RPS, _RPS)])
    pltpu.sync_copy(zer1, dd_sh.at[pl.ds(s * _RPS, _RPS)])
    pltpu.sync_copy(ones_h, ones_v)
    pltpu.sync_copy(ei.at[0, w], sidx)
    pltpu.sync_copy(ei.at[1, w], didx)
    plsc.subcore_barrier()

    @pl.loop(0, _ENC)
    def _deg(j):
        pltpu.sync_copy(ones_v, ds_sh.at[sidx.at[j]], add=True)
        pltpu.sync_copy(ones_v, dd_sh.at[didx.at[j]], add=True)

    pltpu.sync_copy(dpad.at[s], tidx)
    base = c * _HN

    @pl.loop(0, _ENC2)
    def _tr(j):
        for k in range(_ECH2 // 16):
            v = tidx[j, pl.ds(k * 16, 16)]
            v2 = v - base
            bad = jnp.logical_or(v2 < 0, v2 >= _HN)
            spread = _HN + jnp.bitwise_and(v, 1023)
            tidx[j, pl.ds(k * 16, 16)] = jnp.where(bad, spread, v2)

    pltpu.sync_copy(tidx, didx_t.at[c, s])

    pltpu.sync_copy(nidp.at[w], gidx)

    @pl.loop(0, _NNC)
    def _ga(j):
        pltpu.async_copy(emb.at[gidx.at[j]], grows, sem).wait()
        pltpu.sync_copy(grows, xa.at[pl.ds(w * _NPW + j * _NCH, _NCH)])

    pltpu.sync_copy(zp.at[w], gidx)

    @pl.loop(0, _NNC)
    def _gb(j):
        pltpu.async_copy(ztab.at[gidx.at[j]], grows, sem).wait()
        pltpu.sync_copy(grows, xb.at[pl.ds(w * _NPW + j * _NCH, _NCH)])

    plsc.subcore_barrier()
    pltpu.sync_copy(ds_sh.at[pl.ds(s * _RPS, _RPS)],
                    degs.at[c, 0, pl.ds(s * _RPS, _RPS)])
    pltpu.sync_copy(dd_sh.at[pl.ds(s * _RPS, _RPS)],
                    degs.at[c, 1, pl.ds(s * _RPS, _RPS)])


@functools.partial(
    pl.kernel,
    out_type=jax.ShapeDtypeStruct((_NC, _HN, _D), jnp.float32),
    mesh=_sc_mesh,
    scratch_types=[
        pltpu.VMEM((_ENC2, _ECH2), jnp.int32),
        pltpu.VMEM((_ENC2, _ECH2), jnp.int32),
        pltpu.VMEM((_ECH2, _D), jnp.float32),
        pltpu.VMEM((_ECH2, _D), jnp.float32),
        pltpu.VMEM_SHARED((_ACC, _D), jnp.float32),
        pltpu.SemaphoreType.DMA,
        pltpu.SemaphoreType.DMA,
    ],
)
def _sc_prop(h, spad, didx_t, zer128, accp,
             sidx, didx, rows0, rows1, acc_sh, sem0, sem1):
    c = lax.axis_index("c")
    s = lax.axis_index("s")

    @pl.loop(0, _AZS // 64)
    def _zero(k):
        pltpu.sync_copy(zer128, acc_sh.at[pl.ds(s * _AZS + k * 64, 64)])

    pltpu.sync_copy(spad.at[s], sidx)
    pltpu.sync_copy(didx_t.at[c, s], didx)
    plsc.subcore_barrier()

    pltpu.async_copy(h.at[sidx.at[0]], rows0, sem0).wait()

    @pl.loop(0, _ENC2 - 1)
    def _edges(j):
        even = j % 2 == 0

        @pl.when(even)
        def _():
            pltpu.async_copy(h.at[sidx.at[j + 1]], rows1, sem1).wait()
            pltpu.sync_copy(rows0, acc_sh.at[didx.at[j]], add=True)

        @pl.when(jnp.logical_not(even))
        def _():
            pltpu.async_copy(h.at[sidx.at[j + 1]], rows0, sem0).wait()
            pltpu.sync_copy(rows1, acc_sh.at[didx.at[j]], add=True)

    last_even = (_ENC2 - 1) % 2 == 0

    @pl.when(last_even)
    def _le():
        pltpu.sync_copy(rows0, acc_sh.at[didx.at[_ENC2 - 1]], add=True)

    @pl.when(not last_even)
    def _lo():
        pltpu.sync_copy(rows1, acc_sh.at[didx.at[_ENC2 - 1]], add=True)

    plsc.subcore_barrier()
    pltpu.sync_copy(acc_sh.at[pl.ds(s * _XPS, _XPS)],
                    accp.at[c, pl.ds(s * _XPS, _XPS)])


def _norms(degs):
    ns = lax.rsqrt(jnp.maximum((degs[0, 0] + degs[1, 0])[:, None], 1.0))
    nd = lax.rsqrt(jnp.maximum((degs[0, 1] + degs[1, 1])[:, None], 1.0))
    return ns, nd


def _tc_h0_body(xa_ref, xb_ref, pa_ref, pb_ref, degs_ref, h0_ref):
    ns, _ = _norms(degs_ref[...])
    xa = xa_ref[...]
    xb = xb_ref[...]
    ea = jnp.where(pa_ref[...][:, None] == 1, xa[:, 64:], xa[:, :64])
    eb = jnp.where(pb_ref[...][:, None] == 1, xb[:, 64:], xb[:, :64])
    h0_ref[...] = jnp.concatenate([ea, eb], axis=1) * ns


def _tc_layer_body(accp_ref, degs_ref, w_ref, b_ref, h_ref):
    ns, nd = _norms(degs_ref[...])
    agg = jnp.concatenate([accp_ref[0], accp_ref[1]], axis=0) * nd
    o = jnp.dot(agg, w_ref[...], preferred_element_type=jnp.float32) + b_ref[...]
    h_ref[...] = jnp.maximum(o, 0.0) * ns


def _tc_final_body(accp_ref, degs_ref, gid_ref, w3_ref, b3_ref,
                   l1w_ref, l1b_ref, l2w_ref, l2b_ref, out_ref):
    _, nd = _norms(degs_ref[...])
    agg = jnp.concatenate([accp_ref[0], accp_ref[1]], axis=0) * nd
    gio = lax.broadcasted_iota(jnp.int32, (_G, _NP), 0)
    onehot = (gid_ref[...][None, :] == gio).astype(jnp.float32)
    pooled_agg = jnp.dot(onehot, agg, preferred_element_type=jnp.float32)
    counts = jnp.sum(onehot, axis=1, keepdims=True)
    pooled = (jnp.dot(pooled_agg, w3_ref[...], preferred_element_type=jnp.float32)
              + counts * b3_ref[...][None, :])
    hh = jnp.maximum(
        jnp.dot(pooled, l1w_ref[...], preferred_element_type=jnp.float32)
        + l1b_ref[...][None, :], 0.0)
    out_ref[...] = (jnp.dot(hh, l2w_ref[...], preferred_element_type=jnp.float32)
                    + l2b_ref[...][None, :])


def _tc_h0(xa, xb, pa, pb, degs):
    return pl.pallas_call(
        _tc_h0_body,
        out_shape=jax.ShapeDtypeStruct((_NP, _D), jnp.float32),
    )(xa, xb, pa, pb, degs)


def _tc_layer(accp, degs, w, b):
    return pl.pallas_call(
        _tc_layer_body,
        out_shape=jax.ShapeDtypeStruct((_NP, _D), jnp.float32),
    )(accp, degs, w, b)


def _tc_final(accp, degs, gid, w3, b3, l1w, l1b, l2w, l2b):
    return pl.pallas_call(
        _tc_final_body,
        out_shape=jax.ShapeDtypeStruct((_G, 200), jnp.float32),
    )(accp, degs, gid, w3, b3, l1w, l1b, l2w, l2b)


def kernel(edge_index, nid, z, graph_ids, init_embed, z_table,
           W1, b1, W2, b2, W3, b3, l1W, l1b, l2W, l2b):
    ei32 = edge_index.astype(jnp.int32)
    ei = ei32.reshape(2, _NW, _ENC, _ECH)
    spad = jnp.pad(ei32[0], (0, _EP - _E)).reshape(_NS, _ENC2, _ECH2)
    dpad = jnp.pad(ei32[1], (0, _EP - _E),
                   constant_values=1 << 20).reshape(_NS, _ENC2, _ECH2)
    nid32 = nid.astype(jnp.int32)
    z32 = z.astype(jnp.int32)
    emb2 = init_embed.reshape(-1, _D)
    ztab2 = z_table.reshape(-1, _D)
    nidp = jnp.pad(nid32 // 2, (0, _NP - _N)).reshape(_NW, _NNC, _NCH)
    zp = jnp.pad(z32 // 2, (0, _NP - _N)).reshape(_NW, _NNC, _NCH)
    pa = jnp.pad(nid32 % 2, (0, _NP - _N))
    pb = jnp.pad(z32 % 2, (0, _NP - _N))
    ones_h = jnp.ones((_ECH,), jnp.float32)
    zer1 = jnp.zeros((_RPS,), jnp.float32)
    zer128 = jnp.zeros((64, _D), jnp.float32)

    degs, xa, xb, didx_t = _sc_prologue(ei, nidp, zp, dpad, emb2, ztab2,
                                        ones_h, zer1)
    h0 = _tc_h0(xa, xb, pa, pb, degs)
    accp1 = _sc_prop(h0, spad, didx_t, zer128)
    h1 = _tc_layer(accp1, degs, W1, b1)
    accp2 = _sc_prop(h1, spad, didx_t, zer128)
    h2 = _tc_layer(accp2, degs, W2, b2)
    accp3 = _sc_prop(h2, spad, didx_t, zer128)
    gidp = jnp.pad(graph_ids.astype(jnp.int32), (0, _NP - _N),
                   constant_values=_G)
    return _tc_final(accp3, degs, gidp, W3, b3, l1W, l1b, l2W, l2b)


# speedup vs baseline: 2.2837x; 1.1090x over previous
import functools

import jax
import jax.numpy as jnp
from jax import lax
from jax.experimental import pallas as pl
from jax.experimental.pallas import tpu as pltpu
from jax.experimental.pallas import tpu_sc as plsc

_N = 10000
_E = 320000
_G = 64
_D = 128
_NC = 2
_NS = 16
_NW = _NC * _NS
_EPW = _E // _NW
_ECH = 125
_ENC = _EPW // _ECH
_NP = 10240
_NPW = _NP // _NW
_NCH = 80
_NNC = _NPW // _NCH
_RPS = _NP // _NS
_HN = _NP // _NC
_EP = 327680
_ECH2 = 128
_ENC2 = _EP // _NS // _ECH2
_ACC = _HN + 1024
_AZS = _ACC // _NS
_XPS = _HN // _NS

_sc_mesh = plsc.VectorSubcoreMesh(core_axis_name="c", subcore_axis_name="s")


@functools.partial(
    pl.kernel,
    out_type=(
        jax.ShapeDtypeStruct((_NC, 2, _NP), jnp.float32),
        jax.ShapeDtypeStruct((_NP, _D), jnp.float32),
        jax.ShapeDtypeStruct((_NP, _D), jnp.float32),
        jax.ShapeDtypeStruct((_NC, _NS, _ENC2, _ECH2), jnp.int32),
    ),
    mesh=_sc_mesh,
    scratch_types=[
        pltpu.VMEM((_ENC, _ECH), jnp.int32),
        pltpu.VMEM((_ENC, _ECH), jnp.int32),
        pltpu.VMEM((_NNC, _NCH), jnp.int32),
        pltpu.VMEM((_NCH, _D), jnp.float32),
        pltpu.VMEM((_ECH,), jnp.float32),
        pltpu.VMEM((_ENC2, _ECH2), jnp.int32),
        pltpu.VMEM_SHARED((_NP,), jnp.float32),
        pltpu.VMEM_SHARED((_NP,), jnp.float32),
        pltpu.SemaphoreType.DMA,
    ],
)
def _sc_prologue(ei, nidp, zp, dpad, emb, ztab, ones_h, zer1,
                 degs, xa, xb, didx_t,
                 sidx, didx, gidx, grows, ones_v, tidx, ds_sh, dd_sh, sem):
    c = lax.axis_index("c")
    s = lax.axis_index("s")
    w = c * _NS + s
    pltpu.sync_copy(zer1, ds_sh.at[pl.ds(s * _RPS, _RPS)])
    pltpu.sync_copy(zer1, dd_sh.at[pl.ds(s * _RPS, _RPS)])
    pltpu.sync_copy(ones_h, ones_v)
    pltpu.sync_copy(ei.at[0, w], sidx)
    pltpu.sync_copy(ei.at[1, w], didx)
    plsc.subcore_barrier()

    @pl.loop(0, _ENC)
    def _deg(j):
        pltpu.sync_copy(ones_v, ds_sh.at[sidx.at[j]], add=True)
        pltpu.sync_copy(ones_v, dd_sh.at[didx.at[j]], add=True)

    pltpu.sync_copy(dpad.at[s], tidx)
    base = c * _HN

    @pl.loop(0, _ENC2)
    def _tr(j):
        for k in range(_ECH2 // 16):
            v = tidx[j, pl.ds(k * 16, 16)]
            v2 = v - base
            bad = jnp.logical_or(v2 < 0, v2 >= _HN)
            spread = _HN + jnp.bitwise_and(v, 1023)
            tidx[j, pl.ds(k * 16, 16)] = jnp.where(bad, spread, v2)

    pltpu.sync_copy(tidx, didx_t.at[c, s])

    pltpu.sync_copy(nidp.at[w], gidx)

    @pl.loop(0, _NNC)
    def _ga(j):
        pltpu.async_copy(emb.at[gidx.at[j]], grows, sem).wait()
        pltpu.sync_copy(grows, xa.at[pl.ds(w * _NPW + j * _NCH, _NCH)])

    pltpu.sync_copy(zp.at[w], gidx)

    @pl.loop(0, _NNC)
    def _gb(j):
        pltpu.async_copy(ztab.at[gidx.at[j]], grows, sem).wait()
        pltpu.sync_copy(grows, xb.at[pl.ds(w * _NPW + j * _NCH, _NCH)])

    plsc.subcore_barrier()
    pltpu.sync_copy(ds_sh.at[pl.ds(s * _RPS, _RPS)],
                    degs.at[c, 0, pl.ds(s * _RPS, _RPS)])
    pltpu.sync_copy(dd_sh.at[pl.ds(s * _RPS, _RPS)],
                    degs.at[c, 1, pl.ds(s * _RPS, _RPS)])


_NBUF = 2


@functools.partial(
    pl.kernel,
    out_type=jax.ShapeDtypeStruct((_NC, _HN, _D), jnp.float32),
    mesh=_sc_mesh,
    scratch_types=[
        pltpu.VMEM((_ENC2, _ECH2), jnp.int32),
        pltpu.VMEM((_ENC2, _ECH2), jnp.int32),
        [pltpu.VMEM((_ECH2, _D), jnp.float32)] * _NBUF,
        pltpu.VMEM_SHARED((_ACC, _D), jnp.float32),
        [pltpu.SemaphoreType.DMA] * _NBUF,
        [pltpu.SemaphoreType.DMA] * _NBUF,
    ],
)
def _sc_prop(h, spad, didx_t, zer128, accp,
             sidx, didx, rows, acc_sh, sg, ss):
    c = lax.axis_index("c")
    s = lax.axis_index("s")

    @pl.loop(0, _AZS // 64)
    def _zero(k):
        pltpu.sync_copy(zer128, acc_sh.at[pl.ds(s * _AZS + k * 64, 64)])

    pltpu.sync_copy(spad.at[s], sidx)
    pltpu.sync_copy(didx_t.at[c, s], didx)
    plsc.subcore_barrier()

    def _wait_gather(b):
        pltpu.make_async_copy(h.at[sidx.at[0]], rows[b], sg[b]).wait()

    def _wait_scatter(b):
        pltpu.make_async_copy(rows[b], acc_sh.at[didx.at[0]], ss[b]).wait()

    pltpu.async_copy(h.at[sidx.at[0]], rows[0], sg[0])

    @pl.loop(0, _ENC2, step=2)
    def _edges(j0):
        @pl.when(j0 > 0)
        def _():
            _wait_scatter(1)

        pltpu.async_copy(h.at[sidx.at[j0 + 1]], rows[1], sg[1])
        _wait_gather(0)
        pltpu.async_copy(rows[0], acc_sh.at[didx.at[j0]], ss[0], add=True)

        @pl.when(j0 + 2 < _ENC2)
        def _():
            _wait_scatter(0)
            pltpu.async_copy(h.at[sidx.at[j0 + 2]], rows[0], sg[0])

        _wait_gather(1)
        pltpu.async_copy(rows[1], acc_sh.at[didx.at[j0 + 1]], ss[1], add=True)

    _wait_scatter(0)
    _wait_scatter(1)
    plsc.subcore_barrier()
    pltpu.sync_copy(acc_sh.at[pl.ds(s * _XPS, _XPS)],
                    accp.at[c, pl.ds(s * _XPS, _XPS)])


def _norms(degs):
    ns = lax.rsqrt(jnp.maximum((degs[0, 0] + degs[1, 0])[:, None], 1.0))
    nd = lax.rsqrt(jnp.maximum((degs[0, 1] + degs[1, 1])[:, None], 1.0))
    return ns, nd


def _tc_h0_body(xa_ref, xb_ref, pa_ref, pb_ref, degs_ref, h0_ref):
    ns, _ = _norms(degs_ref[...])
    xa = xa_ref[...]
    xb = xb_ref[...]
    ea = jnp.where(pa_ref[...][:, None] == 1, xa[:, 64:], xa[:, :64])
    eb = jnp.where(pb_ref[...][:, None] == 1, xb[:, 64:], xb[:, :64])
    h0_ref[...] = jnp.concatenate([ea, eb], axis=1) * ns


def _tc_layer_body(accp_ref, degs_ref, w_ref, b_ref, h_ref):
    ns, nd = _norms(degs_ref[...])
    agg = jnp.concatenate([accp_ref[0], accp_ref[1]], axis=0) * nd
    o = jnp.dot(agg, w_ref[...], preferred_element_type=jnp.float32) + b_ref[...]
    h_ref[...] = jnp.maximum(o, 0.0) * ns


def _tc_final_body(accp_ref, degs_ref, gid_ref, w3_ref, b3_ref,
                   l1w_ref, l1b_ref, l2w_ref, l2b_ref, out_ref):
    _, nd = _norms(degs_ref[...])
    agg = jnp.concatenate([accp_ref[0], accp_ref[1]], axis=0) * nd
    gio = lax.broadcasted_iota(jnp.int32, (_G, _NP), 0)
    onehot = (gid_ref[...][None, :] == gio).astype(jnp.float32)
    pooled_agg = jnp.dot(onehot, agg, preferred_element_type=jnp.float32)
    counts = jnp.sum(onehot, axis=1, keepdims=True)
    pooled = (jnp.dot(pooled_agg, w3_ref[...], preferred_element_type=jnp.float32)
              + counts * b3_ref[...][None, :])
    hh = jnp.maximum(
        jnp.dot(pooled, l1w_ref[...], preferred_element_type=jnp.float32)
        + l1b_ref[...][None, :], 0.0)
    out_ref[...] = (jnp.dot(hh, l2w_ref[...], preferred_element_type=jnp.float32)
                    + l2b_ref[...][None, :])


def _tc_h0(xa, xb, pa, pb, degs):
    return pl.pallas_call(
        _tc_h0_body,
        out_shape=jax.ShapeDtypeStruct((_NP, _D), jnp.float32),
    )(xa, xb, pa, pb, degs)


def _tc_layer(accp, degs, w, b):
    return pl.pallas_call(
        _tc_layer_body,
        out_shape=jax.ShapeDtypeStruct((_NP, _D), jnp.float32),
    )(accp, degs, w, b)


def _tc_final(accp, degs, gid, w3, b3, l1w, l1b, l2w, l2b):
    return pl.pallas_call(
        _tc_final_body,
        out_shape=jax.ShapeDtypeStruct((_G, 200), jnp.float32),
    )(accp, degs, gid, w3, b3, l1w, l1b, l2w, l2b)


def kernel(edge_index, nid, z, graph_ids, init_embed, z_table,
           W1, b1, W2, b2, W3, b3, l1W, l1b, l2W, l2b):
    ei32 = edge_index.astype(jnp.int32)
    ei = ei32.reshape(2, _NW, _ENC, _ECH)
    spad = jnp.pad(ei32[0], (0, _EP - _E)).reshape(_NS, _ENC2, _ECH2)
    dpad = jnp.pad(ei32[1], (0, _EP - _E),
                   constant_values=1 << 20).reshape(_NS, _ENC2, _ECH2)
    nid32 = nid.astype(jnp.int32)
    z32 = z.astype(jnp.int32)
    emb2 = init_embed.reshape(-1, _D)
    ztab2 = z_table.reshape(-1, _D)
    nidp = jnp.pad(nid32 // 2, (0, _NP - _N)).reshape(_NW, _NNC, _NCH)
    zp = jnp.pad(z32 // 2, (0, _NP - _N)).reshape(_NW, _NNC, _NCH)
    pa = jnp.pad(nid32 % 2, (0, _NP - _N))
    pb = jnp.pad(z32 % 2, (0, _NP - _N))
    ones_h = jnp.ones((_ECH,), jnp.float32)
    zer1 = jnp.zeros((_RPS,), jnp.float32)
    zer128 = jnp.zeros((64, _D), jnp.float32)

    degs, xa, xb, didx_t = _sc_prologue(ei, nidp, zp, dpad, emb2, ztab2,
                                        ones_h, zer1)
    h0 = _tc_h0(xa, xb, pa, pb, degs)
    accp1 = _sc_prop(h0, spad, didx_t, zer128)
    h1 = _tc_layer(accp1, degs, W1, b1)
    accp2 = _sc_prop(h1, spad, didx_t, zer128)
    h2 = _tc_layer(accp2, degs, W2, b2)
    accp3 = _sc_prop(h2, spad, didx_t, zer128)
    gidp = jnp.pad(graph_ids.astype(jnp.int32), (0, _NP - _N),
                   constant_values=_G)
    return _tc_final(accp3, degs, gidp, W3, b3, l1W, l1b, l2W, l2b)


# speedup vs baseline: 3.8003x; 1.6641x over previous
import functools

import jax
import jax.numpy as jnp
from jax import lax
from jax.experimental import pallas as pl
from jax.experimental.pallas import tpu as pltpu
from jax.experimental.pallas import tpu_sc as plsc

_N = 10000
_E = 320000
_G = 64
_D = 128
_NC = 2
_NS = 16
_NW = _NC * _NS
_NP = 10240
_NPW = _NP // _NW
_NCH = 80
_NNC = _NPW // _NCH
_RPS = _NP // _NS
_HN = _NP // _NC
_EP = 327680
_ECH2 = 128
_ENC2 = _EP // _NS // _ECH2
_CAP = (_ENC2 + 2) * _ECH2
_ACC = _HN + 1024
_AZS = _ACC // _NS
_XPS = _HN // _NS
_PADV = _NP - 1

_sc_mesh = plsc.VectorSubcoreMesh(core_axis_name="c", subcore_axis_name="s")


@functools.partial(
    pl.kernel,
    out_type=(
        jax.ShapeDtypeStruct((_NC, _NP), jnp.float32),
        jax.ShapeDtypeStruct((_NP, _D), jnp.float32),
        jax.ShapeDtypeStruct((_NP, _D), jnp.float32),
        jax.ShapeDtypeStruct((_NC, _NS, _CAP), jnp.int32),
        jax.ShapeDtypeStruct((_NC, _NS, _CAP), jnp.int32),
        jax.ShapeDtypeStruct((_NC, _NS, 16), jnp.int32),
    ),
    mesh=_sc_mesh,
    compiler_params=pltpu.CompilerParams(needs_layout_passes=False),
    scratch_types=[
        pltpu.VMEM((_ENC2, _ECH2), jnp.int32),
        pltpu.VMEM((_ENC2, _ECH2), jnp.int32),
        pltpu.VMEM((_CAP,), jnp.int32),
        pltpu.VMEM((_CAP,), jnp.int32),
        pltpu.VMEM((16,), jnp.int32),
        pltpu.VMEM((_NNC, _NCH), jnp.int32),
        pltpu.VMEM((_NCH, _D), jnp.float32),
        pltpu.VMEM((_ECH2,), jnp.float32),
        pltpu.VMEM_SHARED((_NP,), jnp.float32),
        pltpu.SemaphoreType.DMA,
    ],
)
def _sc_prologue(spad, dpad, nidp, zp, emb, ztab, ones_h, zer1,
                 degs, xa, xb, slist, dlist, cnts,
                 sv, dv, slv, dlv, cntv, gidx, grows, ones_v, deg_sh, sem):
    c = lax.axis_index("c")
    s = lax.axis_index("s")
    w = c * _NS + s
    pltpu.sync_copy(zer1, deg_sh.at[pl.ds(s * _RPS, _RPS)])
    pltpu.sync_copy(ones_h, ones_v)
    pltpu.sync_copy(spad.at[s], sv)
    pltpu.sync_copy(dpad.at[s], dv)
    plsc.subcore_barrier()

    @pl.loop(0, _ENC2)
    def _deg(j):
        @pl.when(c == 0)
        def _():
            pltpu.sync_copy(ones_v, deg_sh.at[sv.at[j]], add=True)

        @pl.when(c == 1)
        def _():
            pltpu.sync_copy(ones_v, deg_sh.at[dv.at[j]], add=True)

    base = c * _HN
    lane15 = jnp.full((16,), 15, jnp.int32)

    @pl.loop(0, _ENC2, init_carry=jnp.zeros((16,), jnp.int32))
    def _cmp(j, offv):
        for k in range(_ECH2 // 16):
            d16 = dv[j, pl.ds(k * 16, 16)]
            s16 = sv[j, pl.ds(k * 16, 16)]
            dloc = d16 - base
            m = jnp.logical_and(dloc >= 0, dloc < _HN)
            cum = plsc.cumsum(m.astype(jnp.int32))
            pos = offv + cum - 1
            plsc.store_scatter(slv, [pos], s16, mask=m)
            plsc.store_scatter(dlv, [pos], dloc, mask=m)
            offv = offv + cum[lane15]
        return offv

    offv = _cmp
    fl = lax.iota(jnp.int32, 16)
    zfill = jnp.zeros((16,), jnp.int32)
    dfill = _HN + fl

    @pl.loop(0, 16)
    def _fill(t):
        pos = offv + t * 16 + fl
        plsc.store_scatter(slv, [pos], zfill)
        plsc.store_scatter(dlv, [pos], dfill)

    cntv[...] = jnp.maximum(((offv + 255) // 256) * 2, 2)
    pltpu.sync_copy(cntv, cnts.at[c, s])
    pltpu.sync_copy(slv, slist.at[c, s])
    pltpu.sync_copy(dlv, dlist.at[c, s])

    pltpu.sync_copy(nidp.at[w], gidx)

    @pl.loop(0, _NNC)
    def _ga(j):
        pltpu.async_copy(emb.at[gidx.at[j]], grows, sem).wait()
        pltpu.sync_copy(grows, xa.at[pl.ds(w * _NPW + j * _NCH, _NCH)])

    pltpu.sync_copy(zp.at[w], gidx)

    @pl.loop(0, _NNC)
    def _gb(j):
        pltpu.async_copy(ztab.at[gidx.at[j]], grows, sem).wait()
        pltpu.sync_copy(grows, xb.at[pl.ds(w * _NPW + j * _NCH, _NCH)])

    plsc.subcore_barrier()
    pltpu.sync_copy(deg_sh.at[pl.ds(s * _RPS, _RPS)],
                    degs.at[c, pl.ds(s * _RPS, _RPS)])


@functools.partial(
    pl.kernel,
    out_type=jax.ShapeDtypeStruct((_NC, _HN, _D), jnp.float32),
    mesh=_sc_mesh,
    compiler_params=pltpu.CompilerParams(needs_layout_passes=False),
    scratch_types=[
        pltpu.VMEM((_ENC2 + 2, _ECH2), jnp.int32),
        pltpu.VMEM((_ENC2 + 2, _ECH2), jnp.int32),
        pltpu.VMEM((16,), jnp.int32),
        [pltpu.VMEM((_ECH2, _D), jnp.float32)] * 2,
        pltpu.VMEM_SHARED((_ACC, _D), jnp.float32),
        [pltpu.SemaphoreType.DMA] * 2,
        [pltpu.SemaphoreType.DMA] * 2,
    ],
)
def _sc_prop(h, slist, dlist, cnts, zer128, accp,
             sidx, didx, cntv, rows, acc_sh, sg, ss):
    c = lax.axis_index("c")
    s = lax.axis_index("s")

    @pl.loop(0, _AZS // 64)
    def _zero(k):
        pltpu.sync_copy(zer128, acc_sh.at[pl.ds(s * _AZS + k * 64, 64)])

    pltpu.sync_copy(slist.at[c, s], sidx)
    pltpu.sync_copy(dlist.at[c, s], didx)
    pltpu.sync_copy(cnts.at[c, s], cntv)
    nchv = cntv[...]
    plsc.subcore_barrier()

    def _wait_gather(b):
        pltpu.make_async_copy(h.at[sidx.at[0]], rows[b], sg[b]).wait()

    def _wait_scatter(b):
        pltpu.make_async_copy(rows[b], acc_sh.at[didx.at[0]], ss[b]).wait()

    pltpu.async_copy(h.at[sidx.at[0]], rows[0], sg[0])

    @pl.loop(0, _ENC2, step=2)
    def _edges(j0):
        @pl.when(jnp.any(jnp.broadcast_to(j0, (16,)) < nchv))
        def _active():
            @pl.when(j0 > 0)
            def _():
                _wait_scatter(1)

            pltpu.async_copy(h.at[sidx.at[j0 + 1]], rows[1], sg[1])
            _wait_gather(0)
            pltpu.async_copy(rows[0], acc_sh.at[didx.at[j0]], ss[0], add=True)

            @pl.when(jnp.any(jnp.broadcast_to(j0 + 2, (16,)) < nchv))
            def _():
                _wait_scatter(0)
                pltpu.async_copy(h.at[sidx.at[j0 + 2]], rows[0], sg[0])

            _wait_gather(1)
            pltpu.async_copy(rows[1], acc_sh.at[didx.at[j0 + 1]], ss[1],
                             add=True)

    _wait_scatter(0)
    _wait_scatter(1)
    plsc.subcore_barrier()
    pltpu.sync_copy(acc_sh.at[pl.ds(s * _XPS, _XPS)],
                    accp.at[c, pl.ds(s * _XPS, _XPS)])


def _norms(degs):
    ns = lax.rsqrt(jnp.maximum(degs[0][:, None], 1.0))
    nd = lax.rsqrt(jnp.maximum(degs[1][:, None], 1.0))
    return ns, nd


def _tc_h0_body(xa_ref, xb_ref, pa_ref, pb_ref, degs_ref, h0_ref):
    ns, _ = _norms(degs_ref[...])
    xa = xa_ref[...]
    xb = xb_ref[...]
    ea = jnp.where(pa_ref[...][:, None] == 1, xa[:, 64:], xa[:, :64])
    eb = jnp.where(pb_ref[...][:, None] == 1, xb[:, 64:], xb[:, :64])
    h0_ref[...] = jnp.concatenate([ea, eb], axis=1) * ns


def _tc_layer_body(accp_ref, degs_ref, w_ref, b_ref, h_ref):
    ns, nd = _norms(degs_ref[...])
    agg = jnp.concatenate([accp_ref[0], accp_ref[1]], axis=0) * nd
    o = jnp.dot(agg, w_ref[...], preferred_element_type=jnp.float32) + b_ref[...]
    h_ref[...] = jnp.maximum(o, 0.0) * ns


def _tc_final_body(accp_ref, degs_ref, gid_ref, w3_ref, b3_ref,
                   l1w_ref, l1b_ref, l2w_ref, l2b_ref, out_ref):
    _, nd = _norms(degs_ref[...])
    agg = jnp.concatenate([accp_ref[0], accp_ref[1]], axis=0) * nd
    gio = lax.broadcasted_iota(jnp.int32, (_G, _NP), 0)
    onehot = (gid_ref[...][None, :] == gio).astype(jnp.float32)
    pooled_agg = jnp.dot(onehot, agg, preferred_element_type=jnp.float32)
    counts = jnp.sum(onehot, axis=1, keepdims=True)
    pooled = (jnp.dot(pooled_agg, w3_ref[...], preferred_element_type=jnp.float32)
              + counts * b3_ref[...][None, :])
    hh = jnp.maximum(
        jnp.dot(pooled, l1w_ref[...], preferred_element_type=jnp.float32)
        + l1b_ref[...][None, :], 0.0)
    out_ref[...] = (jnp.dot(hh, l2w_ref[...], preferred_element_type=jnp.float32)
                    + l2b_ref[...][None, :])


def _tc_h0(xa, xb, pa, pb, degs):
    return pl.pallas_call(
        _tc_h0_body,
        out_shape=jax.ShapeDtypeStruct((_NP, _D), jnp.float32),
    )(xa, xb, pa, pb, degs)


def _tc_layer(accp, degs, w, b):
    return pl.pallas_call(
        _tc_layer_body,
        out_shape=jax.ShapeDtypeStruct((_NP, _D), jnp.float32),
    )(accp, degs, w, b)


def _tc_final(accp, degs, gid, w3, b3, l1w, l1b, l2w, l2b):
    return pl.pallas_call(
        _tc_final_body,
        out_shape=jax.ShapeDtypeStruct((_G, 200), jnp.float32),
    )(accp, degs, gid, w3, b3, l1w, l1b, l2w, l2b)


def kernel(edge_index, nid, z, graph_ids, init_embed, z_table,
           W1, b1, W2, b2, W3, b3, l1W, l1b, l2W, l2b):
    ei32 = edge_index.astype(jnp.int32)
    spad = jnp.pad(ei32[0], (0, _EP - _E),
                   constant_values=_PADV).reshape(_NS, _ENC2, _ECH2)
    dpad = jnp.pad(ei32[1], (0, _EP - _E),
                   constant_values=_PADV).reshape(_NS, _ENC2, _ECH2)
    nid32 = nid.astype(jnp.int32)
    z32 = z.astype(jnp.int32)
    emb2 = init_embed.reshape(-1, _D)
    ztab2 = z_table.reshape(-1, _D)
    nidp = jnp.pad(nid32 // 2, (0, _NP - _N)).reshape(_NW, _NNC, _NCH)
    zp = jnp.pad(z32 // 2, (0, _NP - _N)).reshape(_NW, _NNC, _NCH)
    pa = jnp.pad(nid32 % 2, (0, _NP - _N))
    pb = jnp.pad(z32 % 2, (0, _NP - _N))
    ones_h = jnp.ones((_ECH2,), jnp.float32)
    zer1 = jnp.zeros((_RPS,), jnp.float32)
    zer128 = jnp.zeros((64, _D), jnp.float32)

    degs, xa, xb, slist, dlist, cnts = _sc_prologue(
        spad, dpad, nidp, zp, emb2, ztab2, ones_h, zer1)
    slist_r = slist.reshape(_NC, _NS, _ENC2 + 2, _ECH2)
    dlist_r = dlist.reshape(_NC, _NS, _ENC2 + 2, _ECH2)
    h0 = _tc_h0(xa, xb, pa, pb, degs)
    accp1 = _sc_prop(h0, slist_r, dlist_r, cnts, zer128)
    h1 = _tc_layer(accp1, degs, W1, b1)
    accp2 = _sc_prop(h1, slist_r, dlist_r, cnts, zer128)
    h2 = _tc_layer(accp2, degs, W2, b2)
    accp3 = _sc_prop(h2, slist_r, dlist_r, cnts, zer128)
    gidp = jnp.pad(graph_ids.astype(jnp.int32), (0, _NP - _N),
                   constant_values=_G)
    return _tc_final(accp3, degs, gidp, W3, b3, l1W, l1b, l2W, l2b)
